# baseline (speedup 1.0000x reference)
import functools

import jax
import jax.numpy as jnp
from jax import lax
from jax.experimental import pallas as pl
from jax.experimental.pallas import tpu as pltpu
from jax.experimental.pallas import tpu_sc as plsc

D_MODEL = 64
LANES = 16
SLOTS = D_MODEL // LANES
SCALE = 8.0

NUM_CORES = 2
NUM_SUBCORES = 16
NUM_WORKERS = NUM_CORES * NUM_SUBCORES

CHUNK = 512


def _build(B):
    assert B % (NUM_WORKERS * CHUNK) == 0
    per_worker = B // NUM_WORKERS
    n_chunks = per_worker // CHUNK

    mesh = plsc.VectorSubcoreMesh(core_axis_name="c", subcore_axis_name="s")

    @functools.partial(
        pl.kernel,
        mesh=mesh,
        compiler_params=pltpu.CompilerParams(use_tc_tiling_on_sc=False),
        out_type=jax.ShapeDtypeStruct((B, D_MODEL), jnp.float32),
        scratch_types=[
            pltpu.VMEM((CHUNK,), jnp.int32),
            pltpu.VMEM((CHUNK, D_MODEL), jnp.float32),
            pltpu.SemaphoreType.DMA,
        ],
    )
    def run(x_hbm, table_hbm, out_hbm, idx_v, rows, sem):
        wid = lax.axis_index("s") * NUM_CORES + lax.axis_index("c")
        base = wid * per_worker

        @pl.loop(0, n_chunks)
        def _chunk(c):
            off = base + c * CHUNK
            pltpu.sync_copy(x_hbm.at[pl.ds(off, CHUNK)], idx_v)
            pltpu.async_copy(table_hbm.at[idx_v], rows, sem).wait()

            @pl.loop(0, CHUNK)
            def _row(r):
                for j in range(SLOTS):
                    sl = pl.ds(j * LANES, LANES)
                    rows[r, sl] = rows[r, sl] * SCALE

            pltpu.sync_copy(rows, out_hbm.at[pl.ds(off, CHUNK)])

    return run


def kernel(x, table):
    B0, H = x.shape
    B = B0 * H
    xf = x.reshape(B).astype(jnp.int32)
    out = _build(B)(xf, table)
    return out.reshape(B0, H, D_MODEL)


# speedup vs baseline: 1.1368x; 1.1368x over previous
import functools

import jax
import jax.numpy as jnp
from jax import lax
from jax.experimental import pallas as pl
from jax.experimental.pallas import tpu as pltpu
from jax.experimental.pallas import tpu_sc as plsc

D_MODEL = 64
LANES = 16
SLOTS = D_MODEL // LANES
SCALE = 8.0

NUM_CORES = 2
NUM_SUBCORES = 16
NUM_WORKERS = NUM_CORES * NUM_SUBCORES

CHUNK = 512


def _build(B):
    assert B % (NUM_WORKERS * 2 * CHUNK) == 0
    per_worker = B // NUM_WORKERS
    n_pairs = per_worker // (2 * CHUNK)

    mesh = plsc.VectorSubcoreMesh(core_axis_name="c", subcore_axis_name="s")

    @functools.partial(
        pl.kernel,
        mesh=mesh,
        compiler_params=pltpu.CompilerParams(use_tc_tiling_on_sc=False),
        out_type=jax.ShapeDtypeStruct((B, D_MODEL), jnp.float32),
        scratch_types=[
            pltpu.VMEM((per_worker,), jnp.int32),
            pltpu.VMEM((CHUNK, D_MODEL), jnp.float32),
            pltpu.VMEM((CHUNK, D_MODEL), jnp.float32),
            pltpu.SemaphoreType.DMA,
            pltpu.SemaphoreType.DMA,
            pltpu.SemaphoreType.DMA,
            pltpu.SemaphoreType.DMA,
        ],
    )
    def run(x_hbm, table_hbm, out_hbm, idx_v, rows0, rows1, sg0, sg1, so0, so1):
        wid = lax.axis_index("s") * NUM_CORES + lax.axis_index("c")
        base = wid * per_worker
        pltpu.sync_copy(x_hbm.at[pl.ds(base, per_worker)], idx_v)

        def gather(c, rows, sem):
            return pltpu.make_async_copy(
                table_hbm.at[idx_v.at[pl.ds(c * CHUNK, CHUNK)]], rows, sem)

        def writeout(c, rows, sem):
            return pltpu.make_async_copy(
                rows, out_hbm.at[pl.ds(base + c * CHUNK, CHUNK)], sem)

        def scale(rows):
            @plsc.parallel_loop(0, CHUNK, unroll=8)
            def _(r):
                for j in range(SLOTS):
                    sl = pl.ds(j * LANES, LANES)
                    rows[r, sl] = rows[r, sl] * SCALE

        gather(0, rows0, sg0).start()

        @pl.loop(0, n_pairs)
        def _pair(p):
            c0 = 2 * p

            gather(c0, rows0, sg0).wait()

            @pl.when(p > 0)
            def _():
                writeout(c0 - 1, rows1, so1).wait()

            gather(c0 + 1, rows1, sg1).start()
            scale(rows0)
            writeout(c0, rows0, so0).start()

            gather(c0 + 1, rows1, sg1).wait()

            @pl.when(p < n_pairs - 1)
            def _():
                writeout(c0, rows0, so0).wait()
                gather(c0 + 2, rows0, sg0).start()

            scale(rows1)
            writeout(c0 + 1, rows1, so1).start()

        writeout(2 * n_pairs - 2, rows0, so0).wait()
        writeout(2 * n_pairs - 1, rows1, so1).wait()

    return run


def kernel(x, table):
    B0, H = x.shape
    B = B0 * H
    xf = x.reshape(B).astype(jnp.int32)
    out = _build(B)(xf, table)
    return out.reshape(B0, H, D_MODEL)


# speedup vs baseline: 1.1801x; 1.0381x over previous
import functools

import jax
import jax.numpy as jnp
from jax import lax
from jax.experimental import pallas as pl
from jax.experimental.pallas import tpu as pltpu
from jax.experimental.pallas import tpu_sc as plsc

D_MODEL = 64
NUM_CORES = 2
NUM_SUBCORES = 16
NUM_WORKERS = NUM_CORES * NUM_SUBCORES
CHUNK = 256


def _build(B):
    per_worker = B // NUM_WORKERS
    n_pairs = per_worker // (2 * CHUNK)

    mesh = plsc.VectorSubcoreMesh(core_axis_name="c", subcore_axis_name="s")

    @functools.partial(
        pl.kernel,
        mesh=mesh,
        compiler_params=pltpu.CompilerParams(use_tc_tiling_on_sc=False),
        out_type=jax.ShapeDtypeStruct((B, D_MODEL), jnp.float32),
        scratch_types=[
            pltpu.VMEM((per_worker,), jnp.int32),
            pltpu.VMEM((CHUNK, D_MODEL), jnp.float32),
            pltpu.VMEM((CHUNK, D_MODEL), jnp.float32),
            pltpu.SemaphoreType.DMA,
            pltpu.SemaphoreType.DMA,
        ],
    )
    def run(x_hbm, table_hbm, out_hbm, idx_v, rows0, rows1, sg0, sg1):
        wid = lax.axis_index("s") * NUM_CORES + lax.axis_index("c")
        base = wid * per_worker
        pltpu.sync_copy(x_hbm.at[pl.ds(base, per_worker)], idx_v)

        def gather(c, rows, sem):
            return pltpu.make_async_copy(
                table_hbm.at[idx_v.at[pl.ds(c * CHUNK, CHUNK)]], rows, sem)

        gather(0, rows0, sg0).start()

        @pl.loop(0, n_pairs)
        def _pair(p):
            c0 = 2 * p
            gather(c0 + 1, rows1, sg1).start()
            gather(c0, rows0, sg0).wait()

            @pl.when(p < n_pairs - 1)
            def _():
                gather(c0 + 2, rows0, sg0).start()

            gather(c0 + 1, rows1, sg1).wait()

        pltpu.sync_copy(rows0, out_hbm.at[pl.ds(base, CHUNK)])

    return run


def kernel(x, table):
    B0, H = x.shape
    B = B0 * H
    xf = x.reshape(B).astype(jnp.int32)
    out = _build(B)(xf, table)
    return out.reshape(B0, H, 64)


# speedup vs baseline: 1.1831x; 1.0026x over previous
import functools

import jax
import jax.numpy as jnp
from jax import lax
from jax.experimental import pallas as pl
from jax.experimental.pallas import tpu as pltpu
from jax.experimental.pallas import tpu_sc as plsc

D_MODEL = 64
LANES = 16
SCALE = 8.0

NUM_CORES = 2
NUM_SUBCORES = 16
NUM_WORKERS = NUM_CORES * NUM_SUBCORES


def _build(B0, H):
    B = B0 * H
    batches_per_worker = B0 // NUM_WORKERS
    chunk = H
    chunk_pad = ((chunk + LANES - 1) // LANES) * LANES
    n_chunks = batches_per_worker
    per_worker = batches_per_worker * H

    mesh = plsc.VectorSubcoreMesh(core_axis_name="c", subcore_axis_name="s", num_cores=NUM_CORES, num_subcores=NUM_SUBCORES)

    @functools.partial(
        pl.kernel,
        mesh=mesh,
        compiler_params=pltpu.CompilerParams(use_tc_tiling_on_sc=True, needs_layout_passes=False),
        out_type=jax.ShapeDtypeStruct((B0, H, D_MODEL), jnp.float32),
        scratch_types=[
            [pltpu.VMEM((chunk_pad,), jnp.int32) for _ in range(2)],
            [pltpu.VMEM((chunk_pad,), jnp.int32) for _ in range(2)],
            [pltpu.VMEM((chunk, 2 * D_MODEL), jnp.float32) for _ in range(2)],
            [pltpu.VMEM((chunk, D_MODEL), jnp.float32) for _ in range(2)],
            [pltpu.SemaphoreType.DMA for _ in range(2)],
            [pltpu.SemaphoreType.DMA for _ in range(2)],
            [pltpu.SemaphoreType.DMA for _ in range(2)],
        ],
    )
    def run(x_hbm, pairs_hbm, out_hbm, idxs, hidxs, gbufs, obufs, sis, sgs, sos):
        wid = lax.axis_index("s") * NUM_CORES + lax.axis_index("c")
        base = wid * per_worker

        def idx_copy(c, p):
            return pltpu.make_async_copy(
                x_hbm.at[pl.ds(base + c * chunk, chunk)],
                idxs[p].at[pl.ds(0, chunk)], sis[p])

        def halve(p):
            @pl.loop(0, chunk_pad // LANES)
            def _(k):
                sl = pl.ds(k * LANES, LANES)
                hidxs[p][sl] = lax.shift_right_logical(idxs[p][sl], 1)

        def gather(p):
            return pltpu.make_async_copy(
                pairs_hbm.at[hidxs[p].at[pl.ds(0, chunk)]], gbufs[p], sgs[p])

        def writeout(c, p):
            return pltpu.make_async_copy(
                obufs[p], out_hbm.at[wid * n_chunks + c], sos[p])

        def select_scale(p):
            gbuf, obuf, idx = gbufs[p], obufs[p], idxs[p]

            @plsc.parallel_loop(0, chunk, unroll=4)
            def _(r):
                rv = lax.broadcast(r, (LANES,))
                iv = plsc.load_gather(idx, [rv])
                hi_mask = (iv & 1) > 0
                for j in range(D_MODEL // LANES):
                    lo = gbuf[r, pl.ds(j * LANES, LANES)]
                    hi = gbuf[r, pl.ds(D_MODEL + j * LANES, LANES)]
                    v = jnp.where(hi_mask, hi, lo) * SCALE
                    obuf[r, pl.ds(j * LANES, LANES)] = v

        idx_copy(0, 0).start()
        idx_copy(0, 0).wait()
        halve(0)
        gather(0).start()
        idx_copy(1, 1).start()

        @pl.loop(0, n_chunks // 2)
        def _pair(h):
            for p in range(2):
                c = 2 * h + p
                q = 1 - p

                gather(p).wait()

                @pl.when(c + 1 < n_chunks)
                def _():
                    idx_copy(c + 1, q).wait()
                    halve(q)
                    gather(q).start()

                @pl.when(c >= 2)
                def _():
                    writeout(c - 2, p).wait()

                select_scale(p)
                writeout(c, p).start()

                @pl.when(c + 2 < n_chunks)
                def _():
                    idx_copy(c + 2, p).start()

        writeout(n_chunks - 2, 0).wait()
        writeout(n_chunks - 1, 1).wait()

    return run


def kernel(x, table):
    B0, H = x.shape
    V = table.shape[0]
    xf = x.reshape(B0 * H).astype(jnp.int32)
    pairs = table.reshape(V // 2, 2 * D_MODEL)
    return _build(B0, H)(xf, pairs)
